# baseline (speedup 1.0000x reference)
import functools

import jax
import jax.numpy as jnp
from jax import lax
from jax.experimental import pallas as pl
from jax.experimental.pallas import tpu as pltpu
from jax.experimental.pallas import tpu_sc as plsc

B, L, V, D = 4096, 200, 100000, 64
N = B * L
NC, NS = 2, 16
NW = NC * NS
ROWS_PER_W = N // NW
CK = 2 * L
NCH = ROWS_PER_W // CK
NBUF = 4
LANES = 16


def _sc_embed(x_flat, token_weight, pos_weight):
    mesh = plsc.VectorSubcoreMesh(core_axis_name="c", subcore_axis_name="s")

    @functools.partial(
        pl.kernel,
        mesh=mesh,
        compiler_params=pltpu.CompilerParams(use_tc_tiling_on_sc=False),
        out_type=jax.ShapeDtypeStruct((N, D), jnp.float32),
        scratch_types=(
            [pltpu.VMEM((CK,), jnp.int32) for _ in range(NBUF)]
            + [pltpu.VMEM((CK, D), jnp.float32) for _ in range(NBUF)]
            + [pltpu.VMEM((L, D), jnp.float32)]
            + [pltpu.SemaphoreType.DMA for _ in range(3 * NBUF)]
        ),
    )
    def k(x_hbm, tok_hbm, pos_hbm, out_hbm, *scratch):
        idx_b = scratch[0:NBUF]
        rows_b = scratch[NBUF:2 * NBUF]
        pos_v = scratch[2 * NBUF]
        si = scratch[2 * NBUF + 1:2 * NBUF + 1 + NBUF]
        sg = scratch[2 * NBUF + 1 + NBUF:2 * NBUF + 1 + 2 * NBUF]
        so = scratch[2 * NBUF + 1 + 2 * NBUF:2 * NBUF + 1 + 3 * NBUF]

        wid = lax.axis_index("s") * NC + lax.axis_index("c")
        base = wid * ROWS_PER_W
        pltpu.sync_copy(pos_hbm, pos_v)

        def fire_idx(c, b):
            pltpu.async_copy(x_hbm.at[pl.ds(base + c * CK, CK)], idx_b[b], si[b])

        def wait_idx(b):
            pltpu.make_async_copy(x_hbm.at[pl.ds(0, CK)], idx_b[b], si[b]).wait()

        def fire_gather(b):
            pltpu.async_copy(tok_hbm.at[idx_b[b]], rows_b[b], sg[b])

        def wait_gather(b):
            pltpu.make_async_copy(tok_hbm.at[pl.ds(0, CK)], rows_b[b], sg[b]).wait()

        def fire_out(c, b):
            pltpu.async_copy(rows_b[b], out_hbm.at[pl.ds(base + c * CK, CK)], so[b])

        def wait_out(b):
            pltpu.make_async_copy(out_hbm.at[pl.ds(0, CK)], rows_b[b], so[b]).wait()

        def add_pos(b):
            rows = rows_b[b]

            def row_body(r, carry):
                for rep in range(CK // L):
                    row = rep * L + r
                    for kk in range(D // LANES):
                        sl = pl.ds(kk * LANES, LANES)
                        rows[row, sl] = rows[row, sl] + pos_v[r, sl]
                return carry

            lax.fori_loop(0, L, row_body, 0)

        for b in range(NBUF):
            fire_idx(b, b)
        for b in range(NBUF - 1):
            wait_idx(b)
            fire_gather(b)

        def ring_body(q, carry):
            for b in range(NBUF):
                c = NBUF * q + b
                nb3 = (b + NBUF - 1) % NBUF
                wait_gather(b)

                @pl.when(c + NBUF - 1 < NCH)
                def _():
                    wait_idx(nb3)

                    @pl.when(c >= 1)
                    def _():
                        wait_out(nb3)

                    fire_gather(nb3)

                @pl.when(c + NBUF < NCH)
                def _():
                    fire_idx(c + NBUF, b)

                add_pos(b)
                fire_out(c, b)
            return carry

        lax.fori_loop(0, NCH // NBUF, ring_body, 0)
        for b in range(NBUF):
            wait_out(b)

    return k(x_flat, token_weight, pos_weight)


def kernel(x, token_weight, pos_weight):
    x_flat = x.reshape(-1).astype(jnp.int32)
    out = _sc_embed(x_flat, token_weight, pos_weight)
    return out.reshape(B, L, D)
